# Initial kernel scaffold; baseline (speedup 1.0000x reference)
import jax
import jax.numpy as jnp
from jax.experimental import pallas as pl


def kernel(x, edge_index, edge_attr, params):
    raise NotImplementedError("write your pallas kernel here")


# speedup vs baseline: 1.0004x; 1.0004x over previous
import jax
import jax.numpy as jnp
from jax.experimental import pallas as pl
from jax.experimental.pallas import tpu as pltpu


def _final_linear_kernel(h_ref, w_ref, b_ref, o_ref):
    o_ref[...] = h_ref[...] @ w_ref[...] + b_ref[...]


def _final_linear(h, w, b):
    n = h.shape[0]
    return pl.pallas_call(
        _final_linear_kernel,
        out_shape=jax.ShapeDtypeStruct((n, w.shape[1]), jnp.float32),
    )(h, w, b[None, :])


def _gatv2(x, edge_index, edge_attr, p, heads, c):
    n = x.shape[0]
    src, dst = edge_index[0], edge_index[1]
    deg = jnp.zeros((n,), jnp.float32).at[dst].add(1.0)
    loop_attr = jnp.zeros((n, edge_attr.shape[1]), jnp.float32).at[dst].add(edge_attr)
    loop_attr = loop_attr / jnp.clip(deg, 1.0)[:, None]
    ar = jnp.arange(n, dtype=src.dtype)
    src = jnp.concatenate([src, ar])
    dst = jnp.concatenate([dst, ar])
    ea = jnp.concatenate([edge_attr, loop_attr], axis=0)
    xl = (x @ p["Wl"] + p["bl"]).reshape(n, heads, c)
    xr = (x @ p["Wr"] + p["br"]).reshape(n, heads, c)
    e = xl[src] + xr[dst] + (ea @ p["We"]).reshape(-1, heads, c)
    e = jax.nn.leaky_relu(e, 0.2)
    alpha = jnp.sum(e * p["att"], axis=-1)
    amax = jax.ops.segment_max(alpha, dst, num_segments=n)
    alpha = jnp.exp(alpha - amax[dst])
    denom = jax.ops.segment_sum(alpha, dst, num_segments=n)
    alpha = alpha / denom[dst]
    out = jax.ops.segment_sum(xl[src] * alpha[:, :, None], dst, num_segments=n)
    return out.reshape(n, heads * c) + p["bias"]


def _bn(h, g, b):
    return h / jnp.sqrt(1.0 + 1e-5) * g + b


def kernel(x, edge_index, edge_attr, params):
    layer_emb = params["emb_layer"][x[:, 0].astype(jnp.int32)]
    resnet = x[:, 1:1001]
    rel = params["emb_rel"][jnp.round(x[:, 1001] * 10).astype(jnp.int32)]
    col = params["emb_color"][x[:, -3:].astype(jnp.int32)].reshape(x.shape[0], -1)
    h = jnp.concatenate([layer_emb, resnet, rel, col], axis=1)
    h = _gatv2(h, edge_index, edge_attr, params["g1"], 8, 512)
    h = jax.nn.elu(_bn(h, params["bn1g"], params["bn1b"]))
    h = _gatv2(h, edge_index, edge_attr, params["g2"], 8, 256)
    h = jax.nn.elu(_bn(h, params["bn2g"], params["bn2b"]))
    h = _gatv2(h, edge_index, edge_attr, params["g3"], 8, 64)
    h = jax.nn.elu(_bn(h, params["bn3g"], params["bn3b"]))
    h = _gatv2(h, edge_index, edge_attr, params["g4"], 1, 64)
    return _final_linear(h, params["linW"], params["linb"])


# speedup vs baseline: 1.6740x; 1.6734x over previous
import functools
import math

import jax
import jax.numpy as jnp
from jax import lax
from jax.experimental import pallas as pl
from jax.experimental.pallas import tpu as pltpu
from jax.experimental.pallas import tpu_sc as plsc

_NEG = -1e30


def _mm_body(x_ref, w_ref, b_ref, o_ref):
    o_ref[...] = (
        jnp.dot(x_ref[...], w_ref[...], preferred_element_type=jnp.float32)
        + b_ref[...]
    )


def _round_up(v, m):
    return (v + m - 1) // m * m


def _matmul_bias(x, w, b, bm=400, bn=512):
    m, k = x.shape
    n = w.shape[1]
    kp = _round_up(k, 128)
    bn = min(bn, _round_up(n, 128))
    npad = _round_up(n, bn)
    if kp != k:
        x = jnp.pad(x, ((0, 0), (0, kp - k)))
        w = jnp.pad(w, ((0, kp - k), (0, 0)))
    if npad != n:
        w = jnp.pad(w, ((0, 0), (0, npad - n)))
        b = jnp.pad(b, (0, npad - n))
    mp = _round_up(m, bm)
    if mp != m:
        x = jnp.pad(x, ((0, mp - m), (0, 0)))
    out = pl.pallas_call(
        _mm_body,
        grid=(mp // bm, npad // bn),
        in_specs=[
            pl.BlockSpec((bm, kp), lambda i, j: (i, 0)),
            pl.BlockSpec((kp, bn), lambda i, j: (0, j)),
            pl.BlockSpec((1, bn), lambda i, j: (0, j)),
        ],
        out_specs=pl.BlockSpec((bm, bn), lambda i, j: (i, j)),
        out_shape=jax.ShapeDtypeStruct((mp, npad), jnp.float32),
    )(x, w, b[None, :])
    return out[:m, :n]


def _sc_edge_call(n_nodes, heads, c, act, xl, xr, ssrc, sea, nmeta, meta,
                  wev, att, avec, bvec):
    HC = heads * c
    C16 = c // 16
    SLOG = C16.bit_length() - 1
    assert (1 << SLOG) == C16
    K = 8
    KSH = 3

    mesh = plsc.VectorSubcoreMesh(core_axis_name="c", subcore_axis_name="s")

    def body(xl_ref, xr_ref, ssrc_ref, sea_ref, nmeta_ref, meta_ref,
             wev_ref, att_ref, a_ref, b_ref, out_ref,
             idx_v, ea_v, nrow_v, rows_v, z_v, acc_v, o_v, mrow_v,
             wev_v, att_v, a_v, b_v, coef_s, sem):
        lane = lax.iota(jnp.int32, 16)
        wid = lax.axis_index("c") * 16 + lax.axis_index("s")
        pltpu.sync_copy(meta_ref.at[wid], mrow_v)
        pltpu.sync_copy(wev_ref, wev_v)
        pltpu.sync_copy(att_ref, att_v)
        pltpu.sync_copy(a_ref, a_v)
        pltpu.sync_copy(b_ref, b_v)
        mvec = mrow_v[...]
        nstart = mvec[0]
        nend = mvec[1]
        fzero = mvec.astype(jnp.float32) * jnp.float32(0.0)
        negv = fzero - jnp.float32(1e30)

        def node_body(n, carry):
            pltpu.sync_copy(nmeta_ref.at[n], nrow_v)
            nrow = nrow_v[...]
            e0 = nrow[0]
            e1 = nrow[1]
            pltpu.sync_copy(xr_ref.at[n], z_v)

            def zb(j, _):
                acc_v[pl.ds(j * 16, 16)] = fzero
                return 0

            lax.fori_loop(0, heads * C16, zb, 0)

            a0 = pl.multiple_of(jnp.bitwise_and(e0, jnp.int32(-8)), 8)
            nch = lax.shift_right_logical(e1 - a0 + jnp.int32(K - 1), KSH)

            def chunk_body(ci, car):
                c0 = pl.multiple_of(a0 + ci * K, 8)
                pltpu.sync_copy(ssrc_ref.at[pl.ds(c0, K)], idx_v)
                pltpu.sync_copy(sea_ref.at[pl.ds(c0, 16)], ea_v)
                pltpu.async_copy(xl_ref.at[idx_v], rows_v, sem).wait()
                eavec = ea_v[...]
                m, l = car
                for e in range(K):
                    ge = c0 + e
                    process = jnp.logical_and(ge >= e0, ge < e1)
                    ea_s = eavec[e]

                    def hb(h, alpha, e=e, ea_s=ea_s):
                        def ab(j, p, e=e, ea_s=ea_s, h=h):
                            off = ((h << SLOG) + j) * 16
                            sv = (rows_v[e, pl.ds(off, 16)]
                                  + z_v[pl.ds(off, 16)]
                                  + ea_s * wev_v[pl.ds(off, 16)])
                            u = (jnp.float32(0.6) * sv
                                 + jnp.float32(0.4) * jnp.abs(sv))
                            return p + att_v[pl.ds(off, 16)] * u

                        p = lax.fori_loop(0, C16, ab, fzero)
                        ps = _lane_sum(p) + fzero
                        return jnp.where(lane == h, ps, alpha)

                    alpha = lax.fori_loop(0, heads, hb, negv)
                    alpha = jnp.where(process, alpha, negv)
                    newm = jnp.maximum(m, alpha)
                    scl = jnp.exp(m - newm)
                    coef = jnp.exp(alpha - newm)
                    for h in range(heads):
                        coef_s[h] = coef[h]
                        coef_s[heads + h] = scl[h]

                    def cb(j, _, e=e):
                        off = j * 16
                        hh = lax.shift_right_logical(j, SLOG)
                        chh = coef_s[hh]
                        sh = coef_s[heads + hh]
                        acc_v[pl.ds(off, 16)] = (
                            acc_v[pl.ds(off, 16)] * sh
                            + chh * rows_v[e, pl.ds(off, 16)])
                        return 0

                    lax.fori_loop(0, heads * C16, cb, 0)
                    l = l * scl + coef
                    m = newm
                return (m, l)

            m, l = lax.fori_loop(0, nch, chunk_body, (negv, fzero))

            invl = jnp.float32(1.0) / l
            for h in range(heads):
                coef_s[h] = invl[h]

            def fb(j, _):
                off = j * 16
                hh = lax.shift_right_logical(j, SLOG)
                v = acc_v[pl.ds(off, 16)] * coef_s[hh]
                v = v * a_v[pl.ds(off, 16)] + b_v[pl.ds(off, 16)]
                if act:
                    v = jnp.where(v > 0, v, jnp.exp(v) - jnp.float32(1.0))
                o_v[pl.ds(off, 16)] = v
                return 0

            lax.fori_loop(0, heads * C16, fb, 0)
            pltpu.sync_copy(o_v, out_ref.at[n])
            return 0

        lax.fori_loop(nstart, nend, node_body, 0)

    f = pl.kernel(
        body,
        out_type=jax.ShapeDtypeStruct((n_nodes, HC), jnp.float32),
        mesh=mesh,
        scratch_types=[
            pltpu.VMEM((K,), jnp.int32),
            pltpu.VMEM((16,), jnp.float32),
            pltpu.VMEM((16,), jnp.int32),
            pltpu.VMEM((K, HC), jnp.float32),
            pltpu.VMEM((HC,), jnp.float32),
            pltpu.VMEM((HC,), jnp.float32),
            pltpu.VMEM((HC,), jnp.float32),
            pltpu.VMEM((16,), jnp.int32),
            pltpu.VMEM((HC,), jnp.float32),
            pltpu.VMEM((HC,), jnp.float32),
            pltpu.VMEM((HC,), jnp.float32),
            pltpu.VMEM((HC,), jnp.float32),
            pltpu.SMEM((32,), jnp.float32),
            pltpu.SemaphoreType.DMA,
        ],
    )
    return f(xl, xr, ssrc, sea, nmeta, meta, wev, att, avec, bvec)


def _gat_sc(h, pre, p, heads, c, gbn, bbn, act):
    n = h.shape[0]
    HC = heads * c
    xl = _matmul_bias(h, p["Wl"], p["bl"])
    xr = _matmul_bias(h, p["Wr"], p["br"])
    wev = p["We"][0]
    att = p["att"].reshape(HC)
    if act:
        avec = gbn / jnp.sqrt(jnp.float32(1.0 + 1e-5))
        bvec = p["bias"] * avec + bbn
    else:
        avec = jnp.ones((HC,), jnp.float32)
        bvec = p["bias"]
    cp = c
    if HC < 128:
        assert heads == 1
        cp = 128
        pad = 128 - HC
        xl = jnp.pad(xl, ((0, 0), (0, pad)))
        xr = jnp.pad(xr, ((0, 0), (0, pad)))
        wev = jnp.pad(wev, (0, pad))
        att = jnp.pad(att, (0, pad))
        avec = jnp.pad(avec, (0, pad))
        bvec = jnp.pad(bvec, (0, pad))
    out = _sc_edge_call(n, heads, cp, act, xl, xr, pre["ssrc"],
                        pre["sea"], pre["nmeta"], pre["meta"],
                        wev, att, avec, bvec)
    return out[:, :HC]


def _gat_jnp(h, pre, p, heads, c):
    n = h.shape[0]
    src, dst = pre["src_full"], pre["dst_full"]
    ea = pre["ea"]
    xl = _matmul_bias(h, p["Wl"], p["bl"]).reshape(n, heads, c)
    xr = _matmul_bias(h, p["Wr"], p["br"]).reshape(n, heads, c)
    e = xl[src] + xr[dst] + (ea @ p["We"]).reshape(-1, heads, c)
    e = jax.nn.leaky_relu(e, 0.2)
    alpha = jnp.sum(e * p["att"], axis=-1)
    amax = jax.ops.segment_max(alpha, dst, num_segments=n)
    alpha = jnp.exp(alpha - amax[dst])
    denom = jax.ops.segment_sum(alpha, dst, num_segments=n)
    alpha = alpha / denom[dst]
    out = jax.ops.segment_sum(xl[src] * alpha[:, :, None], dst, num_segments=n)
    return out.reshape(n, heads * c) + p["bias"]


def _bn(h, g, b):
    return h / jnp.sqrt(1.0 + 1e-5) * g + b


def _preprocess(n, edge_index, edge_attr):
    src = edge_index[0].astype(jnp.int32)
    dst = edge_index[1].astype(jnp.int32)
    deg = jnp.zeros((n,), jnp.float32).at[dst].add(1.0)
    loop_attr = jnp.zeros((n,), jnp.float32).at[dst].add(edge_attr[:, 0])
    loop_attr = loop_attr / jnp.clip(deg, 1.0)
    ar = jnp.arange(n, dtype=jnp.int32)
    src_full = jnp.concatenate([src, ar])
    dst_full = jnp.concatenate([dst, ar])
    ea_full = jnp.concatenate([edge_attr[:, 0], loop_attr])
    etot = int(src_full.shape[0])
    order = jnp.argsort(dst_full)
    sdst = dst_full[order]
    ssrc = src_full[order]
    sea = ea_full[order]
    rowptr = jnp.searchsorted(sdst, jnp.arange(n + 1, dtype=jnp.int32)
                              ).astype(jnp.int32)
    targets = (jnp.arange(33, dtype=jnp.int32) * etot) // 32
    nstart = jnp.searchsorted(rowptr, targets).astype(jnp.int32)
    meta = jnp.zeros((32, 16), jnp.int32)
    meta = meta.at[:, 0].set(nstart[:32]).at[:, 1].set(nstart[1:])
    pad = 16
    return {
        "src_full": src_full, "dst_full": dst_full,
        "ea": ea_full[:, None],
        "ssrc": jnp.pad(ssrc, (0, pad)),
        "sea": jnp.pad(sea, (0, pad)),
        "nmeta": jnp.zeros((n, 16), jnp.int32)
                 .at[:, 0].set(rowptr[:-1]).at[:, 1].set(rowptr[1:]),
        "meta": meta,
    }


def _lane_splat(v, k):
    dn = lax.GatherDimensionNumbers(offset_dims=(), collapsed_slice_dims=(0,),
                                    start_index_map=(0,))
    idx = jnp.full((16,), k, jnp.int32)
    return lax.gather(v, idx[:, None], dn, (1,),
                      mode=lax.GatherScatterMode.PROMISE_IN_BOUNDS)


def _lane_sum(v):
    dn = lax.GatherDimensionNumbers(offset_dims=(), collapsed_slice_dims=(0,),
                                    start_index_map=(0,))
    lane = lax.iota(jnp.int32, 16)
    for sh in (8, 4, 2, 1):
        idx = jnp.bitwise_xor(lane, sh)
        v = v + lax.gather(v, idx[:, None], dn, (1,),
                           mode=lax.GatherScatterMode.PROMISE_IN_BOUNDS)
    return v


def kernel(x, edge_index, edge_attr, params):
    n = x.shape[0]
    layer_emb = params["emb_layer"][x[:, 0].astype(jnp.int32)]
    resnet = x[:, 1:1001]
    rel = params["emb_rel"][jnp.round(x[:, 1001] * 10).astype(jnp.int32)]
    col = params["emb_color"][x[:, -3:].astype(jnp.int32)].reshape(n, -1)
    h = jnp.concatenate([layer_emb, resnet, rel, col], axis=1)
    pre = _preprocess(n, edge_index, edge_attr)
    h = _gat_sc(h, pre, params["g1"], 8, 512, params["bn1g"], params["bn1b"], True)
    h = _gat_sc(h, pre, params["g2"], 8, 256, params["bn2g"], params["bn2b"], True)
    h = _gat_sc(h, pre, params["g3"], 8, 64, params["bn3g"], params["bn3b"], True)
    h = _gat_sc(h, pre, params["g4"], 1, 64, None, None, False)
    return _matmul_bias(h, params["linW"], params["linb"])


# speedup vs baseline: 1.8032x; 1.0772x over previous
import functools
import math

import jax
import jax.numpy as jnp
from jax import lax
from jax.experimental import pallas as pl
from jax.experimental.pallas import tpu as pltpu
from jax.experimental.pallas import tpu_sc as plsc

_NEG = -1e30


def _mm_body(x_ref, w_ref, b_ref, o_ref):
    o_ref[...] = (
        jnp.dot(x_ref[...], w_ref[...], preferred_element_type=jnp.float32)
        + b_ref[...]
    )


def _round_up(v, m):
    return (v + m - 1) // m * m


def _matmul_bias(x, w, b, bm=400, bn=512):
    m, k = x.shape
    n = w.shape[1]
    kp = _round_up(k, 128)
    bn = min(bn, _round_up(n, 128))
    npad = _round_up(n, bn)
    if kp != k:
        x = jnp.pad(x, ((0, 0), (0, kp - k)))
        w = jnp.pad(w, ((0, kp - k), (0, 0)))
    if npad != n:
        w = jnp.pad(w, ((0, 0), (0, npad - n)))
        b = jnp.pad(b, (0, npad - n))
    mp = _round_up(m, bm)
    if mp != m:
        x = jnp.pad(x, ((0, mp - m), (0, 0)))
    out = pl.pallas_call(
        _mm_body,
        grid=(mp // bm, npad // bn),
        in_specs=[
            pl.BlockSpec((bm, kp), lambda i, j: (i, 0)),
            pl.BlockSpec((kp, bn), lambda i, j: (0, j)),
            pl.BlockSpec((1, bn), lambda i, j: (0, j)),
        ],
        out_specs=pl.BlockSpec((bm, bn), lambda i, j: (i, j)),
        out_shape=jax.ShapeDtypeStruct((mp, npad), jnp.float32),
    )(x, w, b[None, :])
    return out[:m, :n]


def _sc_edge_call(n_nodes, heads, c, act, xl, xr, ssrc, sea, nmeta, meta,
                  wev, att, avec, bvec):
    HC = heads * c
    C16 = c // 16
    SLOG = C16.bit_length() - 1
    assert (1 << SLOG) == C16
    K = 8
    KSH = 3

    mesh = plsc.VectorSubcoreMesh(core_axis_name="c", subcore_axis_name="s")

    def body(xl_ref, xr_ref, ssrc_ref, sea_ref, nmeta_ref, meta_ref,
             wev_ref, att_ref, a_ref, b_ref, out_ref,
             idx_v, ea_v, nrow_v, rows_v, z_v, acc_v, o_v, mrow_v,
             wev_v, att_v, a_v, b_v, coef_s, sem):
        lane = lax.iota(jnp.int32, 16)
        wid = lax.axis_index("c") * 16 + lax.axis_index("s")
        pltpu.sync_copy(meta_ref.at[wid], mrow_v)
        pltpu.sync_copy(wev_ref, wev_v)
        pltpu.sync_copy(att_ref, att_v)
        pltpu.sync_copy(a_ref, a_v)
        pltpu.sync_copy(b_ref, b_v)
        mvec = mrow_v[...]
        nstart = mvec[0]
        nend = mvec[1]
        fzero = mvec.astype(jnp.float32) * jnp.float32(0.0)
        negv = fzero - jnp.float32(1e30)

        def node_body(n, carry):
            pltpu.sync_copy(nmeta_ref.at[n], nrow_v)
            nrow = nrow_v[...]
            e0 = nrow[0]
            e1 = nrow[1]
            pltpu.sync_copy(xr_ref.at[n], z_v)

            def zb(j, _):
                for u4 in range(4):
                    acc_v[pl.ds((j * 4 + u4) * 16, 16)] = fzero
                return 0

            lax.fori_loop(0, heads * C16 // 4, zb, 0)

            a0 = pl.multiple_of(jnp.bitwise_and(e0, jnp.int32(-8)), 8)
            nch = lax.shift_right_logical(e1 - a0 + jnp.int32(K - 1), KSH)

            def chunk_body(ci, car):
                c0 = pl.multiple_of(a0 + ci * K, 8)
                pltpu.sync_copy(ssrc_ref.at[pl.ds(c0, K)], idx_v)
                pltpu.sync_copy(sea_ref.at[pl.ds(c0, 16)], ea_v)
                pltpu.async_copy(xl_ref.at[idx_v], rows_v, sem).wait()
                eavec = ea_v[...]
                m, l = car
                for e in range(K):
                    ge = c0 + e
                    process = jnp.logical_and(ge >= e0, ge < e1)
                    ea_s = eavec[e]

                    def hb(h, alpha, e=e, ea_s=ea_s):
                        def ab(j, p, e=e, ea_s=ea_s, h=h):
                            for u4 in range(4):
                                off = ((h << SLOG) + j * 4 + u4) * 16
                                sv = (rows_v[e, pl.ds(off, 16)]
                                      + z_v[pl.ds(off, 16)]
                                      + ea_s * wev_v[pl.ds(off, 16)])
                                u = (jnp.float32(0.6) * sv
                                     + jnp.float32(0.4) * jnp.abs(sv))
                                p = p + att_v[pl.ds(off, 16)] * u
                            return p

                        p = lax.fori_loop(0, C16 // 4, ab, fzero)
                        ps = _lane_sum(p) + fzero
                        return jnp.where(lane == h, ps, alpha)

                    alpha = lax.fori_loop(0, heads, hb, negv)
                    alpha = jnp.where(process, alpha, negv)
                    newm = jnp.maximum(m, alpha)
                    scl = jnp.exp(m - newm)
                    coef = jnp.exp(alpha - newm)
                    for h in range(heads):
                        coef_s[h] = coef[h]
                        coef_s[heads + h] = scl[h]

                    def cb(j, _, e=e):
                        hh = lax.shift_right_logical(j * 4, SLOG)
                        chh = coef_s[hh]
                        sh = coef_s[heads + hh]
                        for u4 in range(4):
                            off = (j * 4 + u4) * 16
                            acc_v[pl.ds(off, 16)] = (
                                acc_v[pl.ds(off, 16)] * sh
                                + chh * rows_v[e, pl.ds(off, 16)])
                        return 0

                    lax.fori_loop(0, heads * C16 // 4, cb, 0)
                    l = l * scl + coef
                    m = newm
                return (m, l)

            m, l = lax.fori_loop(0, nch, chunk_body, (negv, fzero))

            invl = jnp.float32(1.0) / l
            for h in range(heads):
                coef_s[h] = invl[h]

            def fb(j, _):
                hh = lax.shift_right_logical(j * 4, SLOG)
                ih = coef_s[hh]
                for u4 in range(4):
                    off = (j * 4 + u4) * 16
                    v = acc_v[pl.ds(off, 16)] * ih
                    v = v * a_v[pl.ds(off, 16)] + b_v[pl.ds(off, 16)]
                    if act:
                        v = jnp.where(v > 0, v,
                                      jnp.exp(v) - jnp.float32(1.0))
                    o_v[pl.ds(off, 16)] = v
                return 0

            lax.fori_loop(0, heads * C16 // 4, fb, 0)
            pltpu.sync_copy(o_v, out_ref.at[n])
            return 0

        lax.fori_loop(nstart, nend, node_body, 0)

    f = pl.kernel(
        body,
        out_type=jax.ShapeDtypeStruct((n_nodes, HC), jnp.float32),
        mesh=mesh,
        scratch_types=[
            pltpu.VMEM((K,), jnp.int32),
            pltpu.VMEM((16,), jnp.float32),
            pltpu.VMEM((16,), jnp.int32),
            pltpu.VMEM((K, HC), jnp.float32),
            pltpu.VMEM((HC,), jnp.float32),
            pltpu.VMEM((HC,), jnp.float32),
            pltpu.VMEM((HC,), jnp.float32),
            pltpu.VMEM((16,), jnp.int32),
            pltpu.VMEM((HC,), jnp.float32),
            pltpu.VMEM((HC,), jnp.float32),
            pltpu.VMEM((HC,), jnp.float32),
            pltpu.VMEM((HC,), jnp.float32),
            pltpu.SMEM((32,), jnp.float32),
            pltpu.SemaphoreType.DMA,
        ],
    )
    return f(xl, xr, ssrc, sea, nmeta, meta, wev, att, avec, bvec)


def _gat_sc(h, pre, p, heads, c, gbn, bbn, act):
    n = h.shape[0]
    HC = heads * c
    xl = _matmul_bias(h, p["Wl"], p["bl"])
    xr = _matmul_bias(h, p["Wr"], p["br"])
    wev = p["We"][0]
    att = p["att"].reshape(HC)
    if act:
        avec = gbn / jnp.sqrt(jnp.float32(1.0 + 1e-5))
        bvec = p["bias"] * avec + bbn
    else:
        avec = jnp.ones((HC,), jnp.float32)
        bvec = p["bias"]
    cp = c
    if HC < 128:
        assert heads == 1
        cp = 128
        pad = 128 - HC
        xl = jnp.pad(xl, ((0, 0), (0, pad)))
        xr = jnp.pad(xr, ((0, 0), (0, pad)))
        wev = jnp.pad(wev, (0, pad))
        att = jnp.pad(att, (0, pad))
        avec = jnp.pad(avec, (0, pad))
        bvec = jnp.pad(bvec, (0, pad))
    out = _sc_edge_call(n, heads, cp, act, xl, xr, pre["ssrc"],
                        pre["sea"], pre["nmeta"], pre["meta"],
                        wev, att, avec, bvec)
    return out[:, :HC]


def _gat_jnp(h, pre, p, heads, c):
    n = h.shape[0]
    src, dst = pre["src_full"], pre["dst_full"]
    ea = pre["ea"]
    xl = _matmul_bias(h, p["Wl"], p["bl"]).reshape(n, heads, c)
    xr = _matmul_bias(h, p["Wr"], p["br"]).reshape(n, heads, c)
    e = xl[src] + xr[dst] + (ea @ p["We"]).reshape(-1, heads, c)
    e = jax.nn.leaky_relu(e, 0.2)
    alpha = jnp.sum(e * p["att"], axis=-1)
    amax = jax.ops.segment_max(alpha, dst, num_segments=n)
    alpha = jnp.exp(alpha - amax[dst])
    denom = jax.ops.segment_sum(alpha, dst, num_segments=n)
    alpha = alpha / denom[dst]
    out = jax.ops.segment_sum(xl[src] * alpha[:, :, None], dst, num_segments=n)
    return out.reshape(n, heads * c) + p["bias"]


def _bn(h, g, b):
    return h / jnp.sqrt(1.0 + 1e-5) * g + b


def _preprocess(n, edge_index, edge_attr):
    src = edge_index[0].astype(jnp.int32)
    dst = edge_index[1].astype(jnp.int32)
    deg = jnp.zeros((n,), jnp.float32).at[dst].add(1.0)
    loop_attr = jnp.zeros((n,), jnp.float32).at[dst].add(edge_attr[:, 0])
    loop_attr = loop_attr / jnp.clip(deg, 1.0)
    ar = jnp.arange(n, dtype=jnp.int32)
    src_full = jnp.concatenate([src, ar])
    dst_full = jnp.concatenate([dst, ar])
    ea_full = jnp.concatenate([edge_attr[:, 0], loop_attr])
    etot = int(src_full.shape[0])
    order = jnp.argsort(dst_full)
    sdst = dst_full[order]
    ssrc = src_full[order]
    sea = ea_full[order]
    rowptr = jnp.searchsorted(sdst, jnp.arange(n + 1, dtype=jnp.int32)
                              ).astype(jnp.int32)
    targets = (jnp.arange(33, dtype=jnp.int32) * etot) // 32
    nstart = jnp.searchsorted(rowptr, targets).astype(jnp.int32)
    meta = jnp.zeros((32, 16), jnp.int32)
    meta = meta.at[:, 0].set(nstart[:32]).at[:, 1].set(nstart[1:])
    pad = 16
    return {
        "src_full": src_full, "dst_full": dst_full,
        "ea": ea_full[:, None],
        "ssrc": jnp.pad(ssrc, (0, pad)),
        "sea": jnp.pad(sea, (0, pad)),
        "nmeta": jnp.zeros((n, 16), jnp.int32)
                 .at[:, 0].set(rowptr[:-1]).at[:, 1].set(rowptr[1:]),
        "meta": meta,
    }


def _lane_splat(v, k):
    dn = lax.GatherDimensionNumbers(offset_dims=(), collapsed_slice_dims=(0,),
                                    start_index_map=(0,))
    idx = jnp.full((16,), k, jnp.int32)
    return lax.gather(v, idx[:, None], dn, (1,),
                      mode=lax.GatherScatterMode.PROMISE_IN_BOUNDS)


def _lane_sum(v):
    dn = lax.GatherDimensionNumbers(offset_dims=(), collapsed_slice_dims=(0,),
                                    start_index_map=(0,))
    lane = lax.iota(jnp.int32, 16)
    for sh in (8, 4, 2, 1):
        idx = jnp.bitwise_xor(lane, sh)
        v = v + lax.gather(v, idx[:, None], dn, (1,),
                           mode=lax.GatherScatterMode.PROMISE_IN_BOUNDS)
    return v


def kernel(x, edge_index, edge_attr, params):
    n = x.shape[0]
    layer_emb = params["emb_layer"][x[:, 0].astype(jnp.int32)]
    resnet = x[:, 1:1001]
    rel = params["emb_rel"][jnp.round(x[:, 1001] * 10).astype(jnp.int32)]
    col = params["emb_color"][x[:, -3:].astype(jnp.int32)].reshape(n, -1)
    h = jnp.concatenate([layer_emb, resnet, rel, col], axis=1)
    pre = _preprocess(n, edge_index, edge_attr)
    h = _gat_sc(h, pre, params["g1"], 8, 512, params["bn1g"], params["bn1b"], True)
    h = _gat_sc(h, pre, params["g2"], 8, 256, params["bn2g"], params["bn2b"], True)
    h = _gat_sc(h, pre, params["g3"], 8, 64, params["bn3g"], params["bn3b"], True)
    h = _gat_sc(h, pre, params["g4"], 1, 64, None, None, False)
    return _matmul_bias(h, params["linW"], params["linb"])
